# Initial kernel scaffold; baseline (speedup 1.0000x reference)
import jax
import jax.numpy as jnp
from jax.experimental import pallas as pl


def kernel(hidden_states, Wg, W1, W2, W3):
    raise NotImplementedError("write your pallas kernel here")


# speedup vs baseline: 1.2217x; 1.2217x over previous
import jax
import jax.numpy as jnp
from jax.experimental import pallas as pl
from jax.experimental.pallas import tpu as pltpu

HID = 1024
NE = 8
INTER = 2752
BI = 256
IB = (INTER + BI - 1) // BI
LAST_VALID = INTER - (IB - 1) * BI


def _moe_body(x_ref, wg_ref, w1_ref, w3_ref, w2_ref, out_ref, gate_ref):
    e = pl.program_id(0)
    ib = pl.program_id(1)

    @pl.when((e == 0) & (ib == 0))
    def _init():
        logits = jnp.dot(x_ref[...], wg_ref[...],
                         preferred_element_type=jnp.float32)
        eids = jax.lax.broadcasted_iota(jnp.int32, logits.shape, 1)
        m1 = jnp.max(logits, axis=1, keepdims=True)
        e1 = jnp.min(jnp.where(logits == m1, eids, NE), axis=1, keepdims=True)
        l2m = jnp.where(eids == e1, -jnp.inf, logits)
        m2 = jnp.max(l2m, axis=1, keepdims=True)
        e2 = jnp.min(jnp.where(l2m == m2, eids, NE), axis=1, keepdims=True)
        wa = jax.lax.logistic(m1 - m2)
        gate_ref[...] = (jnp.where(eids == e1, wa, 0.0)
                         + jnp.where(eids == e2, 1.0 - wa, 0.0))
        out_ref[...] = jnp.zeros_like(out_ref)

    x = x_ref[...]
    a = jnp.dot(x, w1_ref[0], preferred_element_type=jnp.float32)
    b = jnp.dot(x, w3_ref[0], preferred_element_type=jnp.float32)
    g = a * jax.lax.logistic(a) * b
    valid = jnp.where(ib == IB - 1, LAST_VALID, BI)
    gcol = jax.lax.broadcasted_iota(jnp.int32, g.shape, 1)
    g = jnp.where(gcol < valid, g, 0.0)
    w2 = w2_ref[0]
    wrow = jax.lax.broadcasted_iota(jnp.int32, w2.shape, 0)
    w2 = jnp.where(wrow < valid, w2, 0.0)
    h = jnp.dot(g, w2, preferred_element_type=jnp.float32)
    gate = gate_ref[...]
    geids = jax.lax.broadcasted_iota(jnp.int32, gate.shape, 1)
    ge = jnp.sum(jnp.where(geids == e, gate, 0.0), axis=1, keepdims=True)
    out_ref[...] += h * ge


def kernel(hidden_states, Wg, W1, W2, W3):
    bs, seq, hid = hidden_states.shape
    x = hidden_states.reshape(-1, hid)
    T = x.shape[0]
    out = pl.pallas_call(
        _moe_body,
        grid=(NE, IB),
        in_specs=[
            pl.BlockSpec((T, HID), lambda e, ib: (0, 0)),
            pl.BlockSpec((HID, NE), lambda e, ib: (0, 0)),
            pl.BlockSpec((1, HID, BI), lambda e, ib: (e, 0, ib)),
            pl.BlockSpec((1, HID, BI), lambda e, ib: (e, 0, ib)),
            pl.BlockSpec((1, BI, HID), lambda e, ib: (e, ib, 0)),
        ],
        out_specs=pl.BlockSpec((T, HID), lambda e, ib: (0, 0)),
        out_shape=jax.ShapeDtypeStruct((T, HID), jnp.float32),
        scratch_shapes=[pltpu.VMEM((T, NE), jnp.float32)],
        compiler_params=pltpu.CompilerParams(
            dimension_semantics=("arbitrary", "arbitrary"),
        ),
    )(x, Wg, W1, W3, W2)
    return out.reshape(bs, seq, hid)
